# baseline (speedup 1.0000x reference)
import jax
import jax.numpy as jnp
from jax.experimental import pallas as pl

_R_MAX = 0.2
_R_MIN = 0.05
_BOUNDARY = 0.1
_K = 64
_NUM_BATCH = 8


def _finish_body(nx_ref, ny_ref, nz_ref, cx_ref, cy_ref, cz_ref, brr_ref, nv_ref,
                 lx_ref, ly_ref, lz_ref, lbw_ref, ins_ref):
    nx = nx_ref[...]
    ny = ny_ref[...]
    nz = nz_ref[...]
    cx = cx_ref[...]
    cy = cy_ref[...]
    cz = cz_ref[...]
    br = (_R_MAX - _R_MIN) * brr_ref[...] + _R_MIN
    lx = (nx - cx) / br
    ly = (ny - cy) / br
    lz = (nz - cz) / br
    norm = jnp.sqrt(lx * lx + ly * ly + lz * lz + 1e-20)
    k_iota = jax.lax.broadcasted_iota(jnp.int32, nx.shape, 1)
    valid = k_iota < nv_ref[...]
    inside = (norm < (1.0 + _BOUNDARY)) & valid
    ratio = (norm - 1.0) / _BOUNDARY
    safe = jnp.where(inside, 1.0 - ratio + 1e-12, 1.0)
    lbw = jnp.where(inside, jnp.log(safe), 0.0)
    lx = jnp.where(inside, lx * br / _R_MAX, 0.0)
    ly = jnp.where(inside, ly * br / _R_MAX, 0.0)
    lz = jnp.where(inside, lz * br / _R_MAX, 0.0)
    lx_ref[...] = lx
    ly_ref[...] = ly
    lz_ref[...] = lz
    lbw_ref[...] = lbw
    ins_ref[...] = inside.astype(jnp.int32)


def _finish(nbr_pos, glimpse_center, ball_radius_ratio, n_valid):
    q = nbr_pos.shape[0]
    f32 = jnp.float32
    outs = pl.pallas_call(
        _finish_body,
        out_shape=(
            jax.ShapeDtypeStruct((q, _K), f32),
            jax.ShapeDtypeStruct((q, _K), f32),
            jax.ShapeDtypeStruct((q, _K), f32),
            jax.ShapeDtypeStruct((q, _K), f32),
            jax.ShapeDtypeStruct((q, _K), jnp.int32),
        ),
    )(
        nbr_pos[:, :, 0], nbr_pos[:, :, 1], nbr_pos[:, :, 2],
        glimpse_center[:, 0:1], glimpse_center[:, 1:2], glimpse_center[:, 2:3],
        ball_radius_ratio, n_valid,
    )
    lx, ly, lz, lbw, ins = outs
    local = jnp.stack([lx, ly, lz], axis=-1)
    return local, lbw, ins.astype(bool)


def kernel(pos, rgb, batch, glimpse_center, voxel_center, ball_radius_ratio,
           center_offset_ratio, glimpse_batch):
    search_r2 = (_R_MAX * (1.0 + _BOUNDARY)) ** 2
    q2 = jnp.sum(glimpse_center * glimpse_center, axis=-1, keepdims=True)
    p2 = jnp.sum(pos * pos, axis=-1)
    qb = glimpse_center.astype(jnp.bfloat16)
    pb = pos.astype(jnp.bfloat16)
    qp = jax.lax.dot_general(
        qb, pb.T, (((1,), (0,)), ((), ())),
        precision=jax.lax.Precision.HIGHEST,
        preferred_element_type=jnp.float32)
    dist2 = q2 + p2[None, :] - 2.0 * qp
    dist2 = jnp.maximum(dist2, 0.0)
    same_batch = glimpse_batch[:, None] == batch[None, :]
    within = (dist2 <= search_r2) & same_batch
    score = jnp.where(within, -dist2, -jnp.inf)
    top_scores, nbr_idx = jax.lax.top_k(score, _K)
    n_valid = jnp.sum(within, axis=1, dtype=jnp.int32)
    n_valid = jnp.minimum(n_valid, _K)[:, None]
    nbr_pos = jnp.take(pos, nbr_idx, axis=0)
    local, lbw, inside = _finish(nbr_pos, glimpse_center, ball_radius_ratio, n_valid)
    return local, lbw, nbr_idx, inside


# speedup vs baseline: 23.9109x; 23.9109x over previous
import functools

import jax
import jax.numpy as jnp
from jax import lax
from jax.experimental import pallas as pl
from jax.experimental.pallas import tpu as pltpu
from jax.experimental.pallas import tpu_sc as plsc

_R_MAX = 0.2
_R_MIN = 0.05
_BOUNDARY = 0.1
_K = 64
_NUM_BATCH = 8
_N = 50000
_Q = 1024
_W = 8192
_NPAD = 57344
_CAND = 1024
_NTILES = 32
_QPT = _Q // _NTILES
_R2 = (_R_MAX * (1.0 + _BOUNDARY)) ** 2
_INF = float("inf")


def _minmax(ka, va, kb, vb):
    m = ka <= kb
    return (jnp.where(m, ka, kb), jnp.where(m, va, vb),
            jnp.where(m, kb, ka), jnp.where(m, vb, va))


def _sc_body(pxb_h, pyb_h, pzb_h, p2_h, batch_h, se_h, qxb_h, qyb_h, qzb_h,
             q2_h, gb_h, posx_h, posy_h, posz_h,
             oidx_h, ox_h, oy_h, oz_h, nv_h,
             win_px, win_py, win_pz, win_p2,
             cand_d, cand_i, f_px, f_py, f_pz, f_p2, f_b, pad_buf,
             qx_v, qy_v, qz_v, q2_v, gb_v, se_v,
             o_idx, g_x, g_y, g_z, nv_v, kd, vd_a, vd_b, sem):
    nc = 2
    wid = lax.axis_index("s") * nc + lax.axis_index("c")
    iota = lax.iota(jnp.int32, 16)

    pltpu.sync_copy(qxb_h.at[pl.ds(wid * _QPT, _QPT)], qx_v.at[pl.ds(0, _QPT)])
    pltpu.sync_copy(qyb_h.at[pl.ds(wid * _QPT, _QPT)], qy_v.at[pl.ds(0, _QPT)])
    pltpu.sync_copy(qzb_h.at[pl.ds(wid * _QPT, _QPT)], qz_v.at[pl.ds(0, _QPT)])
    pltpu.sync_copy(q2_h.at[pl.ds(wid * _QPT, _QPT)], q2_v.at[pl.ds(0, _QPT)])
    pltpu.sync_copy(gb_h.at[pl.ds(wid * _QPT, _QPT)], gb_v.at[pl.ds(0, _QPT)])
    pltpu.sync_copy(se_h, se_v.at[pl.ds(0, 16)])
    pltpu.sync_copy(pxb_h.at[pl.ds(0, 128)], f_px)
    pltpu.sync_copy(pyb_h.at[pl.ds(0, 128)], f_py)
    pltpu.sync_copy(pzb_h.at[pl.ds(0, 128)], f_pz)
    pltpu.sync_copy(p2_h.at[pl.ds(0, 128)], f_p2)
    pltpu.sync_copy(batch_h.at[pl.ds(0, 128)], f_b)
    kd[pl.ds(0, 16)] = jnp.full((16,), -1.0, jnp.float32)
    kd[pl.ds(80, 16)] = jnp.full((16,), -1.0, jnp.float32)

    def query_body(ql, caches):
        ca, cb, nv0, nv1 = caches
        gb = gb_v[pl.ds(ql, 16)][0]
        s = se_v[pl.ds(gb, 16)][0]
        e = se_v[pl.ds(gb + 8, 16)][0]
        qx = qx_v[pl.ds(ql, 16)][0]
        qy = qy_v[pl.ds(ql, 16)][0]
        qz = qz_v[pl.ds(ql, 16)][0]
        q2s = q2_v[pl.ds(ql, 16)][0]
        q64 = ql * _K

        k_lo = s // _W
        k_hi = jnp.maximum(e - 1, s) // _W

        def window_body(k, carry):
            ca, cb, cnt = carry
            sl = k % 2
            cached = jnp.where(sl == 0, ca, cb)

            @pl.when(cached != k)
            def _stage():
                src = pl.ds(k * _W, _W)
                dst = pl.ds(sl * _W, _W)
                pltpu.sync_copy(pxb_h.at[src], win_px.at[dst])
                pltpu.sync_copy(pyb_h.at[src], win_py.at[dst])
                pltpu.sync_copy(pzb_h.at[src], win_pz.at[dst])
                pltpu.sync_copy(p2_h.at[src], win_p2.at[dst])

            ca2 = jnp.where(sl == 0, k, ca)
            cb2 = jnp.where(sl == 1, k, cb)
            lo = jnp.maximum(s, k * _W)
            hi = jnp.minimum(e, (k + 1) * _W)
            c0 = lo & (-16)
            nch = jnp.maximum(hi - c0 + 15, 0) // 16
            lbase = c0 - k * _W + sl * _W

            def chunk_body(i, cnt):
                lb = lbase + i * 16
                px = win_px[pl.ds(lb, 16)]
                py = win_py[pl.ds(lb, 16)]
                pz = win_pz[pl.ds(lb, 16)]
                p2v = win_p2[pl.ds(lb, 16)]
                qp = qx * px + qy * py + qz * pz
                d = (q2s + p2v) - 2.0 * qp
                d = jnp.maximum(d, 0.0)
                g = iota + (c0 + i * 16)
                m = (d <= _R2) & (g >= lo) & (g < hi)
                dk = jnp.where(d == 0.0, g.astype(jnp.float32) * 1e-30, d)
                c2 = jnp.minimum(cnt, _CAND - 16)
                offs = plsc.cumsum(m.astype(jnp.int32))
                posn = jnp.maximum(c2 + offs - 1, 0)
                plsc.store_scatter(cand_d, [posn], dk, mask=m)
                plsc.store_scatter(cand_i, [posn], g, mask=m)
                return cnt + jnp.sum(m.astype(jnp.int32))

            cnt = lax.fori_loop(0, nch, chunk_body, cnt)
            return ca2, cb2, cnt

        ca, cb, cnt = lax.fori_loop(k_lo, k_hi + 1, window_body,
                                    (ca, cb, jnp.int32(0)))

        cs = jnp.minimum(cnt, _CAND)
        nsel = (cs + 15) // 16
        inf16 = jnp.full((16,), _INF, jnp.float32)
        zero16 = jnp.zeros((16,), jnp.int32)

        def sel_body(c, acc):
            a0k, a1k, a2k, a3k, a0v, a1v, a2v, a3v = acc
            base = c * 16
            kc = cand_d[pl.ds(base, 16)]
            vc = cand_i[pl.ds(base, 16)]
            lane = iota + base
            kc = jnp.where(lane < cs, kc, _INF)
            ks, vs = plsc.sort_key_val(kc, vc)
            rk = lax.rev(ks, (0,))
            rv = lax.rev(vs, (0,))
            m3 = a3k <= rk
            l3k = jnp.where(m3, a3k, rk)
            l3v = jnp.where(m3, a3v, rv)
            b0k, b0v, b2k, b2v = _minmax(a0k, a0v, a2k, a2v)
            b1k, b1v, b3k, b3v = _minmax(a1k, a1v, l3k, l3v)
            c0k, c0v, c1k, c1v = _minmax(b0k, b0v, b1k, b1v)
            c2k, c2v, c3k, c3v = _minmax(b2k, b2v, b3k, b3v)
            n0k, n0v = plsc.sort_key_val(c0k, c0v)
            n1k, n1v = plsc.sort_key_val(c1k, c1v)
            n2k, n2v = plsc.sort_key_val(c2k, c2v)
            n3k, n3v = plsc.sort_key_val(c3k, c3v)
            return n0k, n1k, n2k, n3k, n0v, n1v, n2v, n3v

        acc = lax.fori_loop(0, nsel, sel_body,
                            (inf16, inf16, inf16, inf16,
                             zero16, zero16, zero16, zero16))
        for c in range(4):
            kd[pl.ds(16 + c * 16, 16)] = acc[c]
            vd_a[pl.ds(16 + c * 16, 16)] = acc[4 + c]
        vsrc, vdst = vd_a, vd_b
        for p_par in (0, 1, 0):
            for c in range(4):
                b = 16 + c * 16
                posg = iota + c * 16
                if p_par == 0:
                    partner = posg ^ 1
                else:
                    partner = jnp.clip(((posg - 1) ^ 1) + 1, 0, 63)
                k0 = kd[pl.ds(b, 16)]
                v0 = vsrc[pl.ds(b, 16)]
                kp = plsc.load_gather(kd, [partner + 16])
                vp = plsc.load_gather(vsrc, [partner + 16])
                tie = (k0 == kp) & (partner != posg)
                take_min = posg < partner
                nv = jnp.where(
                    tie,
                    jnp.where(take_min, jnp.minimum(v0, vp),
                              jnp.maximum(v0, vp)),
                    v0)
                vdst[pl.ds(b, 16)] = nv
            vsrc, vdst = vdst, vsrc
        for c in range(4):
            o_idx[pl.ds(q64 + c * 16, 16)] = vsrc[pl.ds(16 + c * 16, 16)]
        cnt64 = jnp.minimum(cnt, _K)
        nv0 = jnp.where(iota == ql, cnt64, nv0)
        nv1 = jnp.where(iota + 16 == ql, cnt64, nv1)

        @pl.when(cnt < _K)
        def _pad():
            pcnt = jnp.int32(0)
            for c in range(8):
                g = iota + c * 16
                fx = f_px[pl.ds(c * 16, 16)]
                fy = f_py[pl.ds(c * 16, 16)]
                fz = f_pz[pl.ds(c * 16, 16)]
                fp2 = f_p2[pl.ds(c * 16, 16)]
                fb = f_b[pl.ds(c * 16, 16)]
                qp = qx * fx + qy * fy + qz * fz
                d = (q2s + fp2) - 2.0 * qp
                d = jnp.maximum(d, 0.0)
                inv = jnp.logical_not((fb == gb) & (d <= _R2))
                offs = plsc.cumsum(inv.astype(jnp.int32))
                posn = jnp.maximum(jnp.minimum(pcnt, 128) + offs - 1, 0)
                plsc.store_scatter(pad_buf, [posn], g, mask=inv)
                pcnt = pcnt + jnp.sum(inv.astype(jnp.int32))
            for c in range(4):
                slot = iota + c * 16
                cur = o_idx[pl.ds(q64 + c * 16, 16)]
                srcp = jnp.clip(slot - cnt, 0, 127)
                pv = plsc.load_gather(pad_buf, [srcp])
                o_idx[pl.ds(q64 + c * 16, 16)] = jnp.where(slot >= cnt, pv, cur)

        return ca, cb, nv0, nv1

    zero16i = jnp.zeros((16,), jnp.int32)
    _, _, nv0, nv1 = lax.fori_loop(
        0, _QPT, query_body, (jnp.int32(-1), jnp.int32(-2), zero16i, zero16i))
    nv_v[pl.ds(0, 16)] = nv0
    nv_v[pl.ds(16, 16)] = nv1

    handles = []
    for i in range(16):
        src = o_idx.at[pl.ds(i * 128, 128)]
        dst = pl.ds(i * 128, 128)
        handles.append(pltpu.async_copy(posx_h.at[src], g_x.at[dst], sem))
        handles.append(pltpu.async_copy(posy_h.at[src], g_y.at[dst], sem))
        handles.append(pltpu.async_copy(posz_h.at[src], g_z.at[dst], sem))
    for h in handles:
        h.wait()

    out = pl.ds(wid * (_QPT * _K), _QPT * _K)
    pltpu.sync_copy(o_idx, oidx_h.at[out])
    pltpu.sync_copy(g_x, ox_h.at[out])
    pltpu.sync_copy(g_y, oy_h.at[out])
    pltpu.sync_copy(g_z, oz_h.at[out])
    pltpu.sync_copy(nv_v, nv_h.at[pl.ds(wid * _QPT, _QPT)])


def _sc_search(pxb, pyb, pzb, p2, batch, se, qxb, qyb, qzb, q2, gb,
               posx, posy, posz):
    mesh = plsc.VectorSubcoreMesh(core_axis_name="c", subcore_axis_name="s",
                                  num_cores=2, num_subcores=16)
    f32 = jnp.float32
    i32 = jnp.int32
    kern = pl.kernel(
        _sc_body,
        out_type=(
            jax.ShapeDtypeStruct((_Q * _K,), i32),
            jax.ShapeDtypeStruct((_Q * _K,), f32),
            jax.ShapeDtypeStruct((_Q * _K,), f32),
            jax.ShapeDtypeStruct((_Q * _K,), f32),
            jax.ShapeDtypeStruct((_Q,), i32),
        ),
        mesh=mesh,
        compiler_params=pltpu.CompilerParams(needs_layout_passes=False),
        scratch_types=[
            pltpu.VMEM((2 * _W,), f32),
            pltpu.VMEM((2 * _W,), f32),
            pltpu.VMEM((2 * _W,), f32),
            pltpu.VMEM((2 * _W,), f32),
            pltpu.VMEM((_CAND,), f32),
            pltpu.VMEM((_CAND,), i32),
            pltpu.VMEM((128,), f32),
            pltpu.VMEM((128,), f32),
            pltpu.VMEM((128,), f32),
            pltpu.VMEM((128,), f32),
            pltpu.VMEM((128,), i32),
            pltpu.VMEM((144,), i32),
            pltpu.VMEM((_QPT + 16,), f32),
            pltpu.VMEM((_QPT + 16,), f32),
            pltpu.VMEM((_QPT + 16,), f32),
            pltpu.VMEM((_QPT + 16,), f32),
            pltpu.VMEM((_QPT + 16,), i32),
            pltpu.VMEM((32,), i32),
            pltpu.VMEM((_QPT * _K,), i32),
            pltpu.VMEM((_QPT * _K,), f32),
            pltpu.VMEM((_QPT * _K,), f32),
            pltpu.VMEM((_QPT * _K,), f32),
            pltpu.VMEM((_QPT,), i32),
            pltpu.VMEM((96,), f32),
            pltpu.VMEM((96,), i32),
            pltpu.VMEM((96,), i32),
            pltpu.SemaphoreType.DMA,
        ],
    )
    return kern(pxb, pyb, pzb, p2, batch, se, qxb, qyb, qzb, q2, gb,
                posx, posy, posz)


def _finish_body(nx_ref, ny_ref, nz_ref, cx_ref, cy_ref, cz_ref, brr_ref, nv_ref,
                 lx_ref, ly_ref, lz_ref, lbw_ref, ins_ref):
    nx = nx_ref[...]
    ny = ny_ref[...]
    nz = nz_ref[...]
    cx = cx_ref[...]
    cy = cy_ref[...]
    cz = cz_ref[...]
    br = (_R_MAX - _R_MIN) * brr_ref[...] + _R_MIN
    lx = (nx - cx) / br
    ly = (ny - cy) / br
    lz = (nz - cz) / br
    norm = jnp.sqrt(lx * lx + ly * ly + lz * lz + 1e-20)
    k_iota = jax.lax.broadcasted_iota(jnp.int32, nx.shape, 1)
    valid = k_iota < nv_ref[...]
    inside = (norm < (1.0 + _BOUNDARY)) & valid
    ratio = (norm - 1.0) / _BOUNDARY
    safe = jnp.where(inside, 1.0 - ratio + 1e-12, 1.0)
    lbw = jnp.where(inside, jnp.log(safe), 0.0)
    lx = jnp.where(inside, lx * br / _R_MAX, 0.0)
    ly = jnp.where(inside, ly * br / _R_MAX, 0.0)
    lz = jnp.where(inside, lz * br / _R_MAX, 0.0)
    lx_ref[...] = lx
    ly_ref[...] = ly
    lz_ref[...] = lz
    lbw_ref[...] = lbw
    ins_ref[...] = inside.astype(jnp.int32)


def _finish(nx, ny, nz, glimpse_center, ball_radius_ratio, n_valid):
    f32 = jnp.float32
    outs = pl.pallas_call(
        _finish_body,
        out_shape=(
            jax.ShapeDtypeStruct((_Q, _K), f32),
            jax.ShapeDtypeStruct((_Q, _K), f32),
            jax.ShapeDtypeStruct((_Q, _K), f32),
            jax.ShapeDtypeStruct((_Q, _K), f32),
            jax.ShapeDtypeStruct((_Q, _K), jnp.int32),
        ),
    )(
        nx, ny, nz,
        glimpse_center[:, 0:1], glimpse_center[:, 1:2], glimpse_center[:, 2:3],
        ball_radius_ratio, n_valid,
    )
    lx, ly, lz, lbw, ins = outs
    local = jnp.stack([lx, ly, lz], axis=-1)
    return local, lbw, ins.astype(bool)


def _round_bf16(v):
    u = jax.lax.bitcast_convert_type(v, jnp.uint32)
    r = (u + 0x7FFF + ((u >> 16) & 1)) & jnp.uint32(0xFFFF0000)
    return jax.lax.bitcast_convert_type(r, jnp.float32)


def kernel(pos, rgb, batch, glimpse_center, voxel_center, ball_radius_ratio,
           center_offset_ratio, glimpse_batch):
    f32 = jnp.float32
    i32 = jnp.int32
    posx = pos[:, 0]
    posy = pos[:, 1]
    posz = pos[:, 2]
    padv = jnp.full((_NPAD - _N,), 1e6, f32)
    px_pad = jnp.concatenate([posx, padv])
    py_pad = jnp.concatenate([posy, padv])
    pz_pad = jnp.concatenate([posz, padv])
    p2 = px_pad * px_pad + pz_pad * pz_pad + py_pad * py_pad
    pxb = _round_bf16(px_pad)
    pyb = _round_bf16(py_pad)
    pzb = _round_bf16(pz_pad)
    gx = glimpse_center[:, 0]
    gy = glimpse_center[:, 1]
    gz = glimpse_center[:, 2]
    q2 = gx * gx + gz * gz + gy * gy
    qxb = _round_bf16(gx)
    qyb = _round_bf16(gy)
    qzb = _round_bf16(gz)
    rng = jnp.arange(_NUM_BATCH, dtype=i32)
    starts = jnp.searchsorted(batch, rng, side="left").astype(i32)
    ends = jnp.searchsorted(batch, rng, side="right").astype(i32)
    se = jnp.concatenate([starts, ends])
    gb = glimpse_batch.astype(i32)

    oidx, ox, oy, oz, nv = _sc_search(
        pxb, pyb, pzb, p2, batch.astype(i32), se, qxb, qyb, qzb, q2, gb,
        posx, posy, posz)

    nbr_idx = oidx.reshape(_Q, _K)
    nx = ox.reshape(_Q, _K)
    ny = oy.reshape(_Q, _K)
    nz = oz.reshape(_Q, _K)
    local, lbw, inside = _finish(nx, ny, nz, glimpse_center,
                                 ball_radius_ratio, nv[:, None])
    return local, lbw, nbr_idx, inside
